# baseline (speedup 1.0000x reference)
import jax
import jax.numpy as jnp
from jax import lax
from jax.experimental import pallas as pl

_E = 8
_K = 2
_R = 8
_ALPHA = 32.0
_SCALING = _ALPHA / _R

_BT = 2048


def _moe_body(x_ref, wg_ref, bg_ref, af_ref, bf_ref, rep_ref, o_ref):
    xv = x_ref[...]
    logits_t = lax.dot_general(
        wg_ref[...],
        xv,
        dimension_numbers=(((1,), (1,)), ((), ())),
        preferred_element_type=jnp.float32,
    )
    logits_t = logits_t + bg_ref[...]

    row = lax.broadcasted_iota(jnp.int32, logits_t.shape, 0).astype(jnp.float32)
    neg_inf = jnp.float32(-jnp.inf)
    big = jnp.float32(_E)

    m1 = jnp.max(logits_t, axis=0, keepdims=True)
    i1 = jnp.min(jnp.where(logits_t == m1, row, big), axis=0, keepdims=True)
    sel1 = row == i1
    l2 = jnp.where(sel1, neg_inf, logits_t)
    m2 = jnp.max(l2, axis=0, keepdims=True)
    i2 = jnp.min(jnp.where(l2 == m2, row, big), axis=0, keepdims=True)
    sel2 = row == i2

    p2 = jnp.exp(m2 - m1)
    w1 = 1.0 / (1.0 + p2)
    w2 = 1.0 - w1
    zero = jnp.float32(0.0)
    w_t = jnp.where(sel1, w1, zero) + jnp.where(sel2, w2, zero)

    h = jnp.dot(
        xv.astype(jnp.bfloat16), af_ref[...], preferred_element_type=jnp.float32
    )
    w_rep = lax.dot_general(
        w_t,
        rep_ref[...],
        dimension_numbers=(((0,), (0,)), ((), ())),
        preferred_element_type=jnp.float32,
    )
    g = (h * w_rep).astype(jnp.bfloat16)
    o_ref[...] = jnp.dot(g, bf_ref[...], preferred_element_type=jnp.float32)


@jax.jit
def kernel(x, Wg, bg, A, B):
    T, D = x.shape
    E, R, _ = A.shape
    a_flat = A.reshape(E * R, D).T.astype(jnp.bfloat16)
    b_flat = (
        (B.transpose(0, 2, 1) * jnp.float32(_SCALING))
        .reshape(E * R, D)
        .astype(jnp.bfloat16)
    )
    rep = jnp.repeat(jnp.eye(E, dtype=jnp.float32), R, axis=1)
    bg2 = bg.reshape(E, 1)

    grid = (T // _BT,)
    return pl.pallas_call(
        _moe_body,
        grid=grid,
        in_specs=[
            pl.BlockSpec((_BT, D), lambda i: (i, 0)),
            pl.BlockSpec((E, D), lambda i: (0, 0)),
            pl.BlockSpec((E, 1), lambda i: (0, 0)),
            pl.BlockSpec((D, E * R), lambda i: (0, 0)),
            pl.BlockSpec((E * R, D), lambda i: (0, 0)),
            pl.BlockSpec((E, E * R), lambda i: (0, 0)),
        ],
        out_specs=pl.BlockSpec((_BT, D), lambda i: (i, 0)),
        out_shape=jax.ShapeDtypeStruct((T, D), jnp.float32),
    )(x, Wg, bg2, a_flat, b_flat, rep)


# speedup vs baseline: 1.1299x; 1.1299x over previous
import jax
import jax.numpy as jnp
from jax import lax
from jax.experimental import pallas as pl

_E = 8
_K = 2
_R = 8
_ALPHA = 32.0
_SCALING = _ALPHA / _R

_BT = 2048


def _moe_body(x_ref, wg_ref, bg_ref, af_ref, bf_ref, rep_ref, o_ref):
    xv = x_ref[...]
    logits = jnp.dot(xv, wg_ref[...], preferred_element_type=jnp.float32)
    logits_t = logits.T + bg_ref[...]

    row = lax.broadcasted_iota(jnp.int32, logits_t.shape, 0).astype(jnp.float32)
    neg_inf = jnp.float32(-jnp.inf)
    big = jnp.float32(_E)

    m1 = jnp.max(logits_t, axis=0, keepdims=True)
    i1 = jnp.min(jnp.where(logits_t == m1, row, big), axis=0, keepdims=True)
    sel1 = row == i1
    l2 = jnp.where(sel1, neg_inf, logits_t)
    m2 = jnp.max(l2, axis=0, keepdims=True)
    i2 = jnp.min(jnp.where(l2 == m2, row, big), axis=0, keepdims=True)
    sel2 = row == i2

    p2 = jnp.exp(m2 - m1)
    w1 = 1.0 / (1.0 + p2)
    w2 = 1.0 - w1
    zero = jnp.float32(0.0)
    w_t = jnp.where(sel1, w1, zero) + jnp.where(sel2, w2, zero)

    h = jnp.dot(
        xv.astype(jnp.bfloat16), af_ref[...], preferred_element_type=jnp.float32
    )
    w_rep = lax.dot_general(
        w_t,
        rep_ref[...],
        dimension_numbers=(((0,), (0,)), ((), ())),
        preferred_element_type=jnp.float32,
    )
    g = (h * w_rep).astype(jnp.bfloat16)
    o_ref[...] = jnp.dot(g, bf_ref[...], preferred_element_type=jnp.float32)


@jax.jit
def kernel(x, Wg, bg, A, B):
    T, D = x.shape
    E, R, _ = A.shape
    a_flat = A.reshape(E * R, D).T.astype(jnp.bfloat16)
    b_flat = (
        (B.transpose(0, 2, 1) * jnp.float32(_SCALING))
        .reshape(E * R, D)
        .astype(jnp.bfloat16)
    )
    rep = jnp.repeat(jnp.eye(E, dtype=jnp.float32), R, axis=1)
    bg2 = bg.reshape(E, 1)

    grid = (T // _BT,)
    return pl.pallas_call(
        _moe_body,
        grid=grid,
        in_specs=[
            pl.BlockSpec((_BT, D), lambda i: (i, 0)),
            pl.BlockSpec((D, E), lambda i: (0, 0)),
            pl.BlockSpec((E, 1), lambda i: (0, 0)),
            pl.BlockSpec((D, E * R), lambda i: (0, 0)),
            pl.BlockSpec((E * R, D), lambda i: (0, 0)),
            pl.BlockSpec((E, E * R), lambda i: (0, 0)),
        ],
        out_specs=pl.BlockSpec((_BT, D), lambda i: (i, 0)),
        out_shape=jax.ShapeDtypeStruct((T, D), jnp.float32),
    )(x, Wg.T, bg2, a_flat, b_flat, rep)
